# baseline (speedup 1.0000x reference)
import math

import jax
import jax.numpy as jnp
from jax import lax
from jax.experimental import pallas as pl
from jax.experimental.pallas import tpu as pltpu
from jax.experimental.pallas import tpu_sc as plsc

D_MODEL = 64
SCALE = math.sqrt(D_MODEL)

NC = 2
NS = 16
NW = NC * NS
LANES = 16

X_ROWS = 4096
X_COLS = 200
R_PER_W = X_ROWS // NW
ROWS_UNROLL = 8


def _body(idx_hbm, table_hbm, out_hbm, idx_v, raw0, raw1, scl0, scl1,
          gsem0, gsem1, osem0, osem1):
    c = lax.axis_index("c")
    s = lax.axis_index("s")
    wid = s * NC + c
    base = wid * R_PER_W
    raws = (raw0, raw1)
    scls = (scl0, scl1)
    gsems = (gsem0, gsem1)
    osems = (osem0, osem1)

    pltpu.sync_copy(idx_hbm.at[pl.ds(base, R_PER_W)], idx_v)

    def gather_args(g, b):
        return (table_hbm.at[idx_v.at[g]], raws[b], gsems[b])

    def out_args(g, b):
        return (
            scls[b],
            out_hbm.at[pl.ds((base + g) * X_COLS, X_COLS), pl.ds(0, D_MODEL)],
            osems[b],
        )

    def scale(b):
        raw = raws[b]
        scl = scls[b]

        def rowblk(r, carry):
            for rr in range(ROWS_UNROLL):
                row = r * ROWS_UNROLL + rr
                for d in range(D_MODEL // LANES):
                    sl = pl.ds(d * LANES, LANES)
                    scl[row, sl] = raw[row, sl] * SCALE
            return carry

        lax.fori_loop(0, X_COLS // ROWS_UNROLL, rowblk, 0, unroll=False)

    pltpu.async_copy(*gather_args(0, 0))
    pltpu.async_copy(*gather_args(1, 1))

    def outer(o, carry):
        for b in range(2):
            g = o * 2 + b
            pltpu.make_async_copy(*gather_args(g, b)).wait()

            @pl.when(o >= 1)
            def _():
                pltpu.make_async_copy(*out_args(g - 2, b)).wait()

            scale(b)

            @pl.when(o < (R_PER_W // 2) - 1)
            def _():
                pltpu.async_copy(*gather_args(g + 2, b))

            pltpu.async_copy(*out_args(g, b))
        return carry

    lax.fori_loop(0, R_PER_W // 2, outer, 0, unroll=False)

    for b in range(2):
        pltpu.make_async_copy(*out_args(R_PER_W - 2 + b, b)).wait()


@jax.jit
def kernel(x, table):
    idx = x.astype(jnp.int32)
    table_p = jnp.pad(table, ((0, 0), (0, D_MODEL)))
    mesh = plsc.VectorSubcoreMesh(
        core_axis_name="c", subcore_axis_name="s", num_cores=NC, num_subcores=NS
    )
    out = pl.kernel(
        _body,
        out_type=jax.ShapeDtypeStruct((X_ROWS * X_COLS, 2 * D_MODEL), jnp.float32),
        mesh=mesh,
        scratch_types=[
            pltpu.VMEM((R_PER_W, X_COLS), jnp.int32),
            pltpu.VMEM((X_COLS, 2 * D_MODEL), jnp.float32),
            pltpu.VMEM((X_COLS, 2 * D_MODEL), jnp.float32),
            pltpu.VMEM((X_COLS, D_MODEL), jnp.float32),
            pltpu.VMEM((X_COLS, D_MODEL), jnp.float32),
            pltpu.SemaphoreType.DMA,
            pltpu.SemaphoreType.DMA,
            pltpu.SemaphoreType.DMA,
            pltpu.SemaphoreType.DMA,
        ],
        compiler_params=pltpu.CompilerParams(use_tc_tiling_on_sc=False),
    )(idx, table_p)
    return out[:, :D_MODEL].reshape(X_ROWS, X_COLS, D_MODEL)


# speedup vs baseline: 1.2787x; 1.2787x over previous
import math

import jax
import jax.numpy as jnp
from jax import lax
from jax.experimental import pallas as pl
from jax.experimental.pallas import tpu as pltpu
from jax.experimental.pallas import tpu_sc as plsc

D_MODEL = 64
SCALE = math.sqrt(D_MODEL)

NC = 2
NS = 16
NW = NC * NS
LANES = 16

X_ROWS = 4096
X_COLS = 200
R_PER_W = X_ROWS // NW
ROWS_UNROLL = 8


def _body(idx_hbm, table_hbm, out_hbm, idx_v, raw0, raw1, raw2, raw3,
          scl0, scl1, gsem0, gsem1, gsem2, gsem3, osem0, osem1):
    c = lax.axis_index("c")
    s = lax.axis_index("s")
    wid = s * NC + c
    base = wid * R_PER_W
    raws = (raw0, raw1, raw2, raw3)
    scls = (scl0, scl1)
    gsems = (gsem0, gsem1, gsem2, gsem3)
    osems = (osem0, osem1)

    pltpu.sync_copy(idx_hbm.at[pl.ds(base, R_PER_W)], idx_v)

    def gather_args(g, b):
        return (table_hbm.at[idx_v.at[g]], raws[b], gsems[b])

    def out_args(g, b):
        return (
            scls[b],
            out_hbm.at[pl.ds((base + g) * X_COLS, X_COLS), pl.ds(0, D_MODEL)],
            osems[b],
        )

    def scale(b4, b2):
        raw = raws[b4]
        scl = scls[b2]

        def rowblk(r, carry):
            for rr in range(ROWS_UNROLL):
                row = r * ROWS_UNROLL + rr
                for d in range(D_MODEL // LANES):
                    sl = pl.ds(d * LANES, LANES)
                    scl[row, sl] = raw[row, sl] * SCALE
            return carry

        lax.fori_loop(0, X_COLS // ROWS_UNROLL, rowblk, 0, unroll=False)

    for j in range(4):
        pltpu.async_copy(*gather_args(j, j))

    def outer(o, carry):
        for b4 in range(4):
            g = o * 4 + b4
            b2 = b4 % 2
            pltpu.make_async_copy(*gather_args(g, b4)).wait()

            @pl.when(g >= 2)
            def _():
                pltpu.make_async_copy(*out_args(g - 2, b2)).wait()

            scale(b4, b2)

            @pl.when(o < (R_PER_W // 4) - 1)
            def _():
                pltpu.async_copy(*gather_args(g + 4, b4))

            pltpu.async_copy(*out_args(g, b2))
        return carry

    lax.fori_loop(0, R_PER_W // 4, outer, 0, unroll=False)

    for b in range(2):
        pltpu.make_async_copy(*out_args(R_PER_W - 2 + b, b)).wait()


@jax.jit
def kernel(x, table):
    idx = x.astype(jnp.int32)
    mesh = plsc.VectorSubcoreMesh(
        core_axis_name="c", subcore_axis_name="s", num_cores=NC, num_subcores=NS
    )
    out = pl.kernel(
        _body,
        out_type=jax.ShapeDtypeStruct((X_ROWS * X_COLS, 2 * D_MODEL), jnp.float32),
        mesh=mesh,
        scratch_types=[
            pltpu.VMEM((R_PER_W, X_COLS), jnp.int32),
            pltpu.VMEM((X_COLS, D_MODEL), jnp.float32),
            pltpu.VMEM((X_COLS, D_MODEL), jnp.float32),
            pltpu.VMEM((X_COLS, D_MODEL), jnp.float32),
            pltpu.VMEM((X_COLS, D_MODEL), jnp.float32),
            pltpu.VMEM((X_COLS, D_MODEL), jnp.float32),
            pltpu.VMEM((X_COLS, D_MODEL), jnp.float32),
            pltpu.SemaphoreType.DMA,
            pltpu.SemaphoreType.DMA,
            pltpu.SemaphoreType.DMA,
            pltpu.SemaphoreType.DMA,
            pltpu.SemaphoreType.DMA,
            pltpu.SemaphoreType.DMA,
        ],
        compiler_params=pltpu.CompilerParams(use_tc_tiling_on_sc=False),
    )(idx, table)
    return out[:, :D_MODEL].reshape(X_ROWS, X_COLS, D_MODEL)
